# baseline (speedup 1.0000x reference)
import functools

import jax
import jax.numpy as jnp
from jax import lax
from jax.experimental import pallas as pl
from jax.experimental.pallas import tpu as pltpu
from jax.experimental.pallas import tpu_sc as plsc

B, S, E, C = 4096, 50, 64, 100
SP = 128
NC, NS, L = 2, 16, 16
NW = NC * NS
BPW = B // NW
CB = 8
NCHUNK = BPW // CB
ROWS = CB * S

_mesh = plsc.VectorSubcoreMesh(core_axis_name="c", subcore_axis_name="s")


@functools.partial(
    pl.kernel,
    out_type=jax.ShapeDtypeStruct((B * E,), jnp.float32),
    mesh=_mesh,
    scratch_types=[
        pltpu.VMEM((BPW * SP,), jnp.int32),
        pltpu.VMEM((ROWS, E), jnp.float32),
        pltpu.VMEM((BPW * E,), jnp.float32),
        pltpu.SemaphoreType.DMA,
    ],
    compiler_params=pltpu.CompilerParams(use_tc_tiling_on_sc=False),
)
def _sc_bag_sum(ids_hbm, table_hbm, out_hbm, idx_v, rows_v, acc_v, sem0):
    wid = lax.axis_index("s") * NC + lax.axis_index("c")
    tok_base = wid * (BPW * SP)
    pltpu.sync_copy(ids_hbm.at[pl.ds(tok_base, BPW * SP)], idx_v)

    def chunk_body(c, carry):
        for bi in range(CB):
            idx_sl = idx_v.at[pl.ds((c * CB + bi) * SP, S)]
            pltpu.async_copy(
                table_hbm.at[idx_sl], rows_v.at[pl.ds(bi * S, S), :], sem0
            )
        for bi in range(CB):
            pltpu.make_async_copy(
                table_hbm.at[idx_v.at[pl.ds(bi * SP, S)]],
                rows_v.at[pl.ds(bi * S, S), :],
                sem0,
            ).wait()
        for bi in range(CB):
            rb = bi * S

            def tok_body(j, accs):
                r = rb + j * 2
                accs = tuple(
                    accs[v] + rows_v[r, pl.ds(v * L, L)] for v in range(4)
                )
                return tuple(
                    accs[v] + rows_v[r + 1, pl.ds(v * L, L)] for v in range(4)
                )

            zero = jnp.zeros((L,), jnp.float32)
            accs = lax.fori_loop(0, S // 2, tok_body, (zero,) * 4)
            b = c * CB + bi
            for v in range(4):
                acc_v[pl.ds(b * E + v * L, L)] = accs[v]
        return carry

    lax.fori_loop(0, NCHUNK, chunk_body, 0)
    pltpu.sync_copy(acc_v, out_hbm.at[pl.ds(wid * (BPW * E), BPW * E)])


def _tc_head_body(ids_ref, sums_ref, t01_ref, w_ref, b_ref, out_ref):
    ids = ids_ref[...]
    n0 = jnp.sum((ids == 0).astype(jnp.float32), axis=1, keepdims=True)
    n1 = jnp.sum((ids == 1).astype(jnp.float32), axis=1, keepdims=True)
    agg = (sums_ref[...] - n0 * t01_ref[0:1] - n1 * t01_ref[1:2]) / (
        jnp.float32(S) - n0 - n1
    )
    out_ref[...] = (
        jnp.dot(agg, w_ref[...], preferred_element_type=jnp.float32) + b_ref[...]
    )


@jax.jit
def kernel(input_ids, emb_table, fc_w, fc_b):
    ids = input_ids.astype(jnp.int32)
    ids_pad = jnp.pad(ids, ((0, 0), (0, SP - S)))
    sums = _sc_bag_sum(ids_pad.reshape(-1), emb_table).reshape(B, E)
    logits = pl.pallas_call(
        _tc_head_body,
        out_shape=jax.ShapeDtypeStruct((B, C), jnp.float32),
    )(ids, sums, emb_table[0:2], fc_w, fc_b.reshape(1, C))
    return logits


# speedup vs baseline: 1.5576x; 1.5576x over previous
import functools

import jax
import jax.numpy as jnp
from jax import lax
from jax.experimental import pallas as pl
from jax.experimental.pallas import tpu as pltpu
from jax.experimental.pallas import tpu_sc as plsc

B, S, E, C = 4096, 50, 64, 100
SP = 128
NC, NS, L = 2, 16, 16
NW = NC * NS
BPW = B // NW
CB = 4
NCHUNK = BPW // CB
CTOK = CB * S
CEL = CTOK * E

_mesh = plsc.VectorSubcoreMesh(core_axis_name="c", subcore_axis_name="s")


@functools.partial(
    pl.kernel,
    out_type=jax.ShapeDtypeStruct((B * E,), jnp.float32),
    mesh=_mesh,
    scratch_types=[
        pltpu.VMEM((BPW * SP,), jnp.int32),
        pltpu.VMEM((BPW * E,), jnp.int32),
        pltpu.VMEM((CTOK, E), jnp.float32),
        pltpu.VMEM((CTOK, E), jnp.float32),
        pltpu.VMEM((BPW * E,), jnp.float32),
        pltpu.SemaphoreType.DMA,
        pltpu.SemaphoreType.DMA,
    ],
)
def _sc_bag_sum(
    ids_hbm, table_hbm, out_hbm,
    idx_v, cid_v, rows0, rows1, acc_v, sem_g0, sem_g1,
):
    wid = lax.axis_index("s") * NC + lax.axis_index("c")
    pltpu.sync_copy(ids_hbm.at[pl.ds(wid * (BPW * SP), BPW * SP)], idx_v)

    def compact_body(g, carry):
        for v in range(4):
            cid_v[pl.ds(g * E + v * L, L)] = idx_v[pl.ds(g * SP + v * L, L)]
        return carry

    lax.fori_loop(0, BPW, compact_body, 0)

    def enqueue_chunk(c, rows, sem):
        for bi in range(CB):
            sbase = c * (CB * E) + bi * E
            dbase = bi * S
            for g in range(4):
                n = L if g < 3 else S - 3 * L
                idv = cid_v[pl.ds(sbase + g * L, L)]
                for k in range(n):
                    pltpu.async_copy(
                        table_hbm.at[pl.ds(idv[k], 1), :],
                        rows.at[pl.ds(dbase + g * L + k, 1), :],
                        sem,
                    )

    def drain_chunk(rows, sem):
        pltpu.make_async_copy(table_hbm.at[pl.ds(0, CTOK), :], rows, sem).wait()

    def accum_chunk(c, rows):
        for bi in range(CB):
            rb = bi * S

            def tok_body(j, accs):
                r = rb + j * 2
                accs = tuple(
                    accs[v] + rows[r, pl.ds(v * L, L)] for v in range(4)
                )
                return tuple(
                    accs[v] + rows[r + 1, pl.ds(v * L, L)] for v in range(4)
                )

            zero = jnp.zeros((L,), jnp.float32)
            accs = lax.fori_loop(0, S // 2, tok_body, (zero,) * 4)
            bout = (c * CB + bi) * E
            for v in range(4):
                acc_v[pl.ds(bout + v * L, L)] = accs[v]

    enqueue_chunk(0, rows0, sem_g0)

    def pair_body(p, carry):
        c0 = p * 2
        more = c0 + 2 < NCHUNK

        enqueue_chunk(c0 + 1, rows1, sem_g1)

        drain_chunk(rows0, sem_g0)
        accum_chunk(c0, rows0)

        @pl.when(more)
        def _():
            enqueue_chunk(c0 + 2, rows0, sem_g0)

        drain_chunk(rows1, sem_g1)
        accum_chunk(c0 + 1, rows1)
        return carry

    lax.fori_loop(0, NCHUNK // 2, pair_body, 0)
    pltpu.sync_copy(acc_v, out_hbm.at[pl.ds(wid * (BPW * E), BPW * E)])


def _tc_head_body(ids_ref, sums_ref, t01_ref, w_ref, b_ref, out_ref):
    ids = ids_ref[...]
    n0 = jnp.sum((ids == 0).astype(jnp.float32), axis=1, keepdims=True)
    n1 = jnp.sum((ids == 1).astype(jnp.float32), axis=1, keepdims=True)
    agg = (sums_ref[...] - n0 * t01_ref[0:1] - n1 * t01_ref[1:2]) / (
        jnp.float32(S) - n0 - n1
    )
    out_ref[...] = (
        jnp.dot(agg, w_ref[...], preferred_element_type=jnp.float32) + b_ref[...]
    )


@jax.jit
def kernel(input_ids, emb_table, fc_w, fc_b):
    ids = input_ids.astype(jnp.int32)
    ids_pad = jnp.pad(ids, ((0, 0), (0, SP - S)))
    sums = _sc_bag_sum(ids_pad.reshape(-1), emb_table).reshape(B, E)
    logits = pl.pallas_call(
        _tc_head_body,
        out_shape=jax.ShapeDtypeStruct((B, C), jnp.float32),
    )(ids, sums, emb_table[0:2], fc_w, fc_b.reshape(1, C))
    return logits
